# baseline (device time: 33554 ns/iter reference)
import jax
import jax.numpy as jnp
from jax import lax
from jax.experimental import pallas as pl
from jax.experimental.pallas import tpu as pltpu

N_Z = 4
B, H, D, BS = 8, 8, 64, 16
PAGES_PER_SHARD = 64
KEYS = PAGES_PER_SHARD * BS
NSLOT = 64
SCALE = D ** -0.5
NEG = -1e30


def kernel(Q, K, V, bt, lens):
    K_t = jnp.transpose(K.reshape(KEYS, H, D), (1, 0, 2))
    V_t = jnp.transpose(V.reshape(KEYS, H, D), (1, 0, 2))
    bt_T = bt.T

    def body(q_ref, k_ref, v_ref, btT_ref, lens_ref, out_ref,
             comm_ref, send_sems, recv_sems):
        my_x = lax.axis_index("x")
        my_y = lax.axis_index("y")
        my_z = lax.axis_index("z")

        barrier_sem = pltpu.get_barrier_semaphore()
        for d in (1, 2, 3):
            pl.semaphore_signal(
                barrier_sem, inc=1,
                device_id=(my_x, my_y, (my_z + d) % N_Z),
                device_id_type=pl.DeviceIdType.MESH,
            )
        pl.semaphore_wait(barrier_sem, 3)

        z_off = my_z * PAGES_PER_SHARD
        pid = z_off + lax.broadcasted_iota(jnp.int32, (NSLOT, KEYS), 1) // BS
        slot_iota = lax.broadcasted_iota(jnp.int32, (NSLOT, KEYS), 0)

        for b in range(B):
            bt_col = btT_ref[:, b:b + 1]
            used = jnp.logical_and(bt_col == pid, slot_iota < lens_ref[b])
            w = jnp.sum(jnp.where(used, 1.0, 0.0), axis=0, keepdims=True)
            has = w > 0.0
            for h in range(H):
                q = q_ref[b, 0, h:h + 1, :]
                s = lax.dot_general(
                    q, k_ref[h], (((1,), (1,)), ((), ())),
                    preferred_element_type=jnp.float32) * SCALE
                s = jnp.where(has, s, NEG)
                m = jnp.max(s, axis=1, keepdims=True)
                e = jnp.exp(s - m) * w
                l = jnp.sum(e, axis=1, keepdims=True)
                o = lax.dot_general(
                    e, v_ref[h], (((1,), (0,)), ((), ())),
                    preferred_element_type=jnp.float32)
                comm_ref[0, 0, b, h:h + 1, :] = o
                comm_ref[0, 1, b, h:h + 1, :] = jnp.broadcast_to(m, (1, D))
                comm_ref[0, 2, b, h:h + 1, :] = jnp.broadcast_to(l, (1, D))

        rdmas = []
        for d in (1, 2, 3):
            dst_slot = N_Z - d
            rdma = pltpu.make_async_remote_copy(
                src_ref=comm_ref.at[0],
                dst_ref=comm_ref.at[dst_slot],
                send_sem=send_sems.at[d - 1],
                recv_sem=recv_sems.at[dst_slot - 1],
                device_id=(my_x, my_y, (my_z + d) % N_Z),
                device_id_type=pl.DeviceIdType.MESH,
            )
            rdma.start()
            rdmas.append(rdma)
        for rdma in rdmas:
            rdma.wait_recv()
        for rdma in rdmas:
            rdma.wait_send()

        o_all = comm_ref[:, 0]
        m_all = comm_ref[:, 1]
        l_all = comm_ref[:, 2]
        m_max = jnp.max(m_all, axis=0)
        alpha = jnp.exp(m_all - m_max[None])
        l_tot = jnp.sum(l_all * alpha, axis=0)
        out_ref[:, 0, :, :] = jnp.sum(o_all * alpha, axis=0) / l_tot

    return pl.pallas_call(
        body,
        out_shape=jax.ShapeDtypeStruct((B, 1, H, D), jnp.float32),
        in_specs=[
            pl.BlockSpec(memory_space=pltpu.VMEM),
            pl.BlockSpec(memory_space=pltpu.VMEM),
            pl.BlockSpec(memory_space=pltpu.VMEM),
            pl.BlockSpec(memory_space=pltpu.VMEM),
            pl.BlockSpec(memory_space=pltpu.SMEM),
        ],
        out_specs=pl.BlockSpec(memory_space=pltpu.VMEM),
        scratch_shapes=[
            pltpu.VMEM((N_Z, 3, B, H, D), jnp.float32),
            pltpu.SemaphoreType.DMA((3,)),
            pltpu.SemaphoreType.DMA((3,)),
        ],
        compiler_params=pltpu.CompilerParams(collective_id=0),
    )(Q, K_t, V_t, bt_T, lens)


# device time: 18171 ns/iter; 1.8466x vs baseline; 1.8466x over previous
import jax
import jax.numpy as jnp
from jax import lax
from jax.experimental import pallas as pl
from jax.experimental.pallas import tpu as pltpu

N_Z = 4
B, H, D, BS = 8, 8, 64, 16
PAGES_PER_SHARD = 64
KEYS = PAGES_PER_SHARD * BS
NSLOT = 64
SCALE = D ** -0.5
NEG = -1e30


def kernel(Q, K, V, bt, lens):
    K_t = jnp.transpose(K.reshape(KEYS, H, D), (1, 0, 2))
    V_t = jnp.transpose(V.reshape(KEYS, H, D), (1, 0, 2))
    Q_t = jnp.transpose(Q[:, 0, :, :], (1, 0, 2))
    bt_T = bt.T

    def body(q_ref, k_ref, v_ref, btT_ref, lens_ref, out_ref,
             comm_ref, w_ref, send_sems, recv_sems):
        my_x = lax.axis_index("x")
        my_y = lax.axis_index("y")
        my_z = lax.axis_index("z")

        z_off = my_z * PAGES_PER_SHARD
        pid = z_off + lax.broadcasted_iota(jnp.int32, (NSLOT, KEYS), 1) // BS
        slot_iota = lax.broadcasted_iota(jnp.int32, (NSLOT, KEYS), 0)

        for b in range(B):
            bt_col = btT_ref[:, b:b + 1]
            used = jnp.logical_and(bt_col == pid, slot_iota < lens_ref[b])
            w_ref[b:b + 1, :] = jnp.sum(
                jnp.where(used, 1.0, 0.0), axis=0, keepdims=True)

        w = w_ref[:, :]
        has = w > 0.0
        for h in range(H):
            s = lax.dot_general(
                q_ref[h], k_ref[h], (((1,), (1,)), ((), ())),
                preferred_element_type=jnp.float32) * SCALE
            s = jnp.where(has, s, NEG)
            m = jnp.max(s, axis=1, keepdims=True)
            e = jnp.exp(s - m) * w
            l = jnp.sum(e, axis=1, keepdims=True)
            o = lax.dot_general(
                e, v_ref[h], (((1,), (0,)), ((), ())),
                preferred_element_type=jnp.float32)
            comm_ref[0, 0, :, h, :] = o
            comm_ref[0, 1, :, h, :] = jnp.broadcast_to(m, (B, D))
            comm_ref[0, 2, :, h, :] = jnp.broadcast_to(l, (B, D))

        barrier_sem = pltpu.get_barrier_semaphore()
        for d in (1, 2, 3):
            pl.semaphore_signal(
                barrier_sem, inc=1,
                device_id=(my_x, my_y, (my_z + d) % N_Z),
                device_id_type=pl.DeviceIdType.MESH,
            )
        pl.semaphore_wait(barrier_sem, 3)

        rdmas = []
        for d in (1, 2, 3):
            dst_slot = N_Z - d
            rdma = pltpu.make_async_remote_copy(
                src_ref=comm_ref.at[0],
                dst_ref=comm_ref.at[dst_slot],
                send_sem=send_sems.at[d - 1],
                recv_sem=recv_sems.at[dst_slot - 1],
                device_id=(my_x, my_y, (my_z + d) % N_Z),
                device_id_type=pl.DeviceIdType.MESH,
            )
            rdma.start()
            rdmas.append(rdma)
        for rdma in rdmas:
            rdma.wait_recv()
        for rdma in rdmas:
            rdma.wait_send()

        o_all = comm_ref[:, 0]
        m_all = comm_ref[:, 1]
        l_all = comm_ref[:, 2]
        m_max = jnp.max(m_all, axis=0)
        alpha = jnp.exp(m_all - m_max[None])
        l_tot = jnp.sum(l_all * alpha, axis=0)
        out_ref[:, 0, :, :] = jnp.sum(o_all * alpha, axis=0) / l_tot

    return pl.pallas_call(
        body,
        out_shape=jax.ShapeDtypeStruct((B, 1, H, D), jnp.float32),
        in_specs=[
            pl.BlockSpec(memory_space=pltpu.VMEM),
            pl.BlockSpec(memory_space=pltpu.VMEM),
            pl.BlockSpec(memory_space=pltpu.VMEM),
            pl.BlockSpec(memory_space=pltpu.VMEM),
            pl.BlockSpec(memory_space=pltpu.SMEM),
        ],
        out_specs=pl.BlockSpec(memory_space=pltpu.VMEM),
        scratch_shapes=[
            pltpu.VMEM((N_Z, 3, B, H, D), jnp.float32),
            pltpu.VMEM((B, KEYS), jnp.float32),
            pltpu.SemaphoreType.DMA((3,)),
            pltpu.SemaphoreType.DMA((3,)),
        ],
        compiler_params=pltpu.CompilerParams(collective_id=0),
    )(Q_t, K_t, V_t, bt_T, lens)
